# Initial kernel scaffold; baseline (speedup 1.0000x reference)
import jax
import jax.numpy as jnp
from jax.experimental import pallas as pl


def kernel(x, node_pos, edge_feat4, enc, pos_mlp, ns_mlp, edge_map, node_upd, edge_upd, dec, edge_index):
    raise NotImplementedError("write your pallas kernel here")


# speedup vs baseline: 3.1469x; 3.1469x over previous
import functools

import jax
import jax.numpy as jnp
from jax import lax
from jax.experimental import pallas as pl
from jax.experimental.pallas import tpu as pltpu
from jax.experimental.pallas import tpu_sc as plsc

B = 8
V = 10242
E = 61440
L = 64
R = 16

NC = 2
NS = 16
NW = NC * NS
CH = 128
ZR = 641
VP = NS * ZR

F32 = jnp.float32


def _silu(x):
    return x * (1.0 / (1.0 + jnp.exp(-x)))


def _dot(a, b):
    return jnp.dot(a, b, preferred_element_type=F32)


def _full_spec(shape):
    return pl.BlockSpec(shape, lambda *_: tuple(0 for _ in shape))


def _mlp3_rows(x, w1, b1, w2, b2, w3, b3):
    t = _silu(_dot(x, w1) + b1)
    t = _silu(_dot(t, w2) + b2)
    return _dot(t, w3) + b3


def _enc_body(x_ref, w1, b1, w2, b2, w3, b3, out_ref):
    out_ref[...] = _mlp3_rows(x_ref[...], w1[...], b1[...], w2[...], b2[...],
                              w3[...], b3[...])


def _enc_call(x, ws):
    (w1, b1), (w2, b2), (w3, b3) = ws
    n, _ = x.shape
    return pl.pallas_call(
        _enc_body,
        out_shape=jax.ShapeDtypeStruct((n, w3.shape[1]), F32),
    )(x, w1, b1.reshape(1, -1), w2, b2.reshape(1, -1), w3, b3.reshape(1, -1))


def _rows_mlp_body(x_ref, w1, b1, w2, b2, w3, b3, out_ref):
    out_ref[...] = _mlp3_rows(x_ref[...], w1[...], b1[...], w2[...], b2[...],
                              w3[...], b3[...])


def _rows_mlp_call(x, ws, blk):
    (w1, b1), (w2, b2), (w3, b3) = ws
    n, k = x.shape
    grid = (pl.cdiv(n, blk),)
    return pl.pallas_call(
        _rows_mlp_body,
        grid=grid,
        in_specs=[
            pl.BlockSpec((blk, k), lambda i: (i, 0)),
            _full_spec(w1.shape), _full_spec((1, w1.shape[1])),
            _full_spec(w2.shape), _full_spec((1, w2.shape[1])),
            _full_spec(w3.shape), _full_spec((1, w3.shape[1])),
        ],
        out_specs=pl.BlockSpec((blk, w3.shape[1]), lambda i: (i, 0)),
        out_shape=jax.ShapeDtypeStruct((n, w3.shape[1]), F32),
    )(x, w1, b1.reshape(1, -1), w2, b2.reshape(1, -1), w3, b3.reshape(1, -1))


def _hinit_body(lat_ref, pos_ref, w1x, w1p, b1, w2, b2, w3, b3, out_ref):
    b = pl.program_id(0)
    sel = lax.broadcasted_iota(jnp.int32, (B, L), 0) == b
    xe = jnp.sum(jnp.where(sel, lat_ref[...], 0.0), axis=0, keepdims=True)
    t = _silu(_dot(xe, w1x[...]) + _dot(pos_ref[...], w1p[...]) + b1[...])
    t = _silu(_dot(t, w2[...]) + b2[...])
    out_ref[0] = _dot(t, w3[...]) + b3[...] + xe


def _hinit_call(lat, pos_emb, ws, blk):
    (w1, b1), (w2, b2), (w3, b3) = ws
    w1x, w1p = w1[:L], w1[L:]
    grid = (B, pl.cdiv(V, blk))
    return pl.pallas_call(
        _hinit_body,
        grid=grid,
        in_specs=[
            pl.BlockSpec((B, L), lambda b, v: (0, 0)),
            pl.BlockSpec((blk, L), lambda b, v: (v, 0)),
            _full_spec((L, L)), _full_spec((L, L)), _full_spec((1, L)),
            _full_spec((L, L)), _full_spec((1, L)),
            _full_spec((L, L)), _full_spec((1, L)),
        ],
        out_specs=pl.BlockSpec((1, blk, L), lambda b, v: (b, v, 0)),
        out_shape=jax.ShapeDtypeStruct((B, V, L), F32),
    )(lat, pos_emb, w1x, w1p, b1.reshape(1, -1), w2, b2.reshape(1, -1),
      w3, b3.reshape(1, -1))


def _edge_body(g_ref, ef_ref, w1g, w1e, b1, w2, b2, w3, b3, out_ref):
    ef = ef_ref[...]
    t = _silu(_dot(g_ref[...], w1g[...]) + _dot(ef, w1e[...]) + b1[...])
    t = _silu(_dot(t, w2[...]) + b2[...])
    out_ref[...] = _dot(t, w3[...]) + b3[...] + ef


def _edge_call(g, ef, ws, blk):
    (w1, b1), (w2, b2), (w3, b3) = ws
    w1g, w1e = w1[:2 * L], w1[2 * L:]
    grid = (E // blk,)
    return pl.pallas_call(
        _edge_body,
        grid=grid,
        in_specs=[
            pl.BlockSpec((blk, 2 * L), lambda i: (i, 0)),
            pl.BlockSpec((blk, L), lambda i: (i, 0)),
            _full_spec((2 * L, L)), _full_spec((L, L)), _full_spec((1, L)),
            _full_spec((L, L)), _full_spec((1, L)),
            _full_spec((L, L)), _full_spec((1, L)),
        ],
        out_specs=pl.BlockSpec((blk, L), lambda i: (i, 0)),
        out_shape=jax.ShapeDtypeStruct((E, L), F32),
    )(g, ef, w1g, w1e, b1.reshape(1, -1), w2, b2.reshape(1, -1),
      w3, b3.reshape(1, -1))


def _node0_body(h_ref, agg_ref, w1h, w1a, b1, w2, b2, w3, b3, out_ref):
    h = h_ref[...]
    agg = agg_ref[0] + agg_ref[1]
    t = _silu(_dot(h, w1h[...]) + _dot(agg, w1a[...]) + b1[...])
    t = _silu(_dot(t, w2[...]) + b2[...])
    out_ref[...] = _dot(t, w3[...]) + b3[...] + h


def _node0_call(h0, agg2, ws, blk):
    (w1, b1), (w2, b2), (w3, b3) = ws
    w1h, w1a = w1[:L], w1[L:]
    grid = (pl.cdiv(V, blk),)
    return pl.pallas_call(
        _node0_body,
        grid=grid,
        in_specs=[
            pl.BlockSpec((blk, L), lambda i: (i, 0)),
            pl.BlockSpec((NC, blk, L), lambda i: (0, i, 0)),
            _full_spec((L, L)), _full_spec((L, L)), _full_spec((1, L)),
            _full_spec((L, L)), _full_spec((1, L)),
            _full_spec((L, L)), _full_spec((1, L)),
        ],
        out_specs=pl.BlockSpec((blk, L), lambda i: (i, 0)),
        out_shape=jax.ShapeDtypeStruct((V, L), F32),
    )(h0, agg2, w1h, w1a, b1.reshape(1, -1), w2, b2.reshape(1, -1),
      w3, b3.reshape(1, -1))


def _hrest_body(h_ref, w1h, b1, w2, b2, w3, b3, out_ref):
    def rnd(_, h):
        t = _silu(_dot(h, w1h[...]) + b1[...])
        t = _silu(_dot(t, w2[...]) + b2[...])
        return _dot(t, w3[...]) + b3[...] + h

    out_ref[...] = lax.fori_loop(0, R, rnd, h_ref[...])


def _hrest_call(h, ws, blk):
    (w1, b1), (w2, b2), (w3, b3) = ws
    w1h = w1[:L]
    n = h.shape[0]
    grid = (pl.cdiv(n, blk),)
    return pl.pallas_call(
        _hrest_body,
        grid=grid,
        in_specs=[
            pl.BlockSpec((blk, L), lambda i: (i, 0)),
            _full_spec((L, L)), _full_spec((1, L)),
            _full_spec((L, L)), _full_spec((1, L)),
            _full_spec((L, L)), _full_spec((1, L)),
        ],
        out_specs=pl.BlockSpec((blk, L), lambda i: (i, 0)),
        out_shape=jax.ShapeDtypeStruct((n, L), F32),
    )(h, w1h, b1.reshape(1, -1), w2, b2.reshape(1, -1), w3, b3.reshape(1, -1))


def _pool_body(h_ref, out_ref):
    m = jnp.sum(h_ref[0], axis=0, keepdims=True) * (1.0 / V)
    out_ref[0] = jnp.broadcast_to(m, (8, L))


def _pool_call(h):
    k = h.shape[0] // V
    out = pl.pallas_call(
        _pool_body,
        grid=(k,),
        in_specs=[pl.BlockSpec((1, V, L), lambda i: (i, 0, 0))],
        out_specs=pl.BlockSpec((1, 8, L), lambda i: (i, 0, 0)),
        out_shape=jax.ShapeDtypeStruct((k, 8, L), F32),
    )(h.reshape(k, V, L))
    return out[:, 0, :]


def _dec_body(x_ref, w1, b1, w2, b2, w3, b3, out_ref):
    t = _silu(_dot(x_ref[...], w1[...]) + b1[...])
    t = _silu(_dot(t, w2[...]) + b2[...])
    out_ref[...] = _dot(t, w3[...]) + b3[...]


def _dec_call(x, ws):
    (w1, b1), (w2, b2), (w3, b3) = ws
    return pl.pallas_call(
        _dec_body,
        out_shape=jax.ShapeDtypeStruct((B, w3.shape[1]), F32),
    )(x, w1, b1.reshape(1, -1), w2, b2.reshape(1, -1), w3, b3.reshape(1, -1))


def _sc_mesh():
    return plsc.VectorSubcoreMesh(core_axis_name="c", subcore_axis_name="s",
                                  num_cores=NC, num_subcores=NS)


def _sc_gather(h0, idx2):
    n_chunks = (2 * E) // (NW * CH)
    per_w = (2 * E) // NW

    @functools.partial(
        pl.kernel,
        out_type=jax.ShapeDtypeStruct((2 * E, L), F32),
        mesh=_sc_mesh(),
        compiler_params=pltpu.CompilerParams(use_tc_tiling_on_sc=False),
        scratch_types=[
            pltpu.VMEM((CH,), jnp.int32),
            pltpu.VMEM((CH, L), F32),
            pltpu.SemaphoreType.DMA,
        ],
    )
    def k(h0_hbm, idx_hbm, out_hbm, idx_v, rows_v, sem):
        wid = lax.axis_index("s") * NC + lax.axis_index("c")
        base = wid * per_w

        def chunk(i, carry):
            off = base + i * CH
            pltpu.sync_copy(idx_hbm.at[pl.ds(off, CH)], idx_v)
            pltpu.async_copy(h0_hbm.at[idx_v], rows_v, sem).wait()
            pltpu.sync_copy(rows_v, out_hbm.at[pl.ds(off, CH)])
            return carry

        lax.fori_loop(0, n_chunks, chunk, 0)

    return k(h0, idx2)


def _sc_scatter(ef, dst):
    n_chunks = E // (NW * CH)
    per_w = E // NW

    @functools.partial(
        pl.kernel,
        out_type=jax.ShapeDtypeStruct((NC, VP, L), F32),
        mesh=_sc_mesh(),
        compiler_params=pltpu.CompilerParams(use_tc_tiling_on_sc=False),
        scratch_types=[
            pltpu.VMEM((CH,), jnp.int32),
            pltpu.VMEM((CH, L), F32),
            pltpu.VMEM((ZR, L), F32),
            pltpu.VMEM_SHARED((VP, L), F32),
            pltpu.SemaphoreType.DMA,
        ],
    )
    def k(ef_hbm, dst_hbm, agg_hbm, idx_v, val_v, zbuf_v, agg_sh, sem):
        c = lax.axis_index("c")
        s = lax.axis_index("s")
        wid = s * NC + c
        base = wid * per_w
        rows0 = s * ZR

        z16 = jnp.zeros((16,), F32)

        def zrow(i, carry):
            for j in range(L // 16):
                zbuf_v[i, pl.ds(j * 16, 16)] = z16
            return carry

        lax.fori_loop(0, ZR, zrow, 0)
        pltpu.sync_copy(zbuf_v, agg_sh.at[pl.ds(rows0, ZR)])
        plsc.subcore_barrier()

        def chunk(i, carry):
            off = base + i * CH
            pltpu.sync_copy(dst_hbm.at[pl.ds(off, CH)], idx_v)
            pltpu.sync_copy(ef_hbm.at[pl.ds(off, CH)], val_v)
            pltpu.sync_copy(val_v, agg_sh.at[idx_v], add=True)
            return carry

        lax.fori_loop(0, n_chunks, chunk, 0)
        plsc.subcore_barrier()

        pltpu.sync_copy(agg_sh.at[pl.ds(rows0, ZR)], zbuf_v)
        pltpu.sync_copy(zbuf_v, agg_hbm.at[c, pl.ds(rows0, ZR)])

    return k(ef, dst)


def kernel(x, node_pos, edge_feat4, enc, pos_mlp, ns_mlp, edge_map,
           node_upd, edge_upd, dec, edge_index):
    lat = _enc_call(x, enc)

    npad = jnp.pad(node_pos, ((0, 0), (0, 5)))
    pw1, pb1 = pos_mlp[0]
    pos_ws = [(jnp.pad(pw1, ((0, 5), (0, 0))), pb1)] + list(pos_mlp[1:])
    pos_emb = _rows_mlp_call(npad, pos_ws, 2048)

    h = _hinit_call(lat, pos_emb, ns_mlp, 2048)
    h0 = h[0]
    hrest = h[1:].reshape((B - 1) * V, L)

    epad = jnp.pad(edge_feat4, ((0, 0), (0, 4)))
    ew1, eb1 = edge_map[0]
    edge_ws = [(jnp.pad(ew1, ((0, 4), (0, 0))), eb1)] + list(edge_map[1:])
    ef = _rows_mlp_call(epad, edge_ws, 2048)

    src = edge_index[0]
    dst = edge_index[1]
    idx2 = jnp.stack([src, dst], axis=1).reshape(2 * E)

    for _ in range(R):
        g = _sc_gather(h0, idx2).reshape(E, 2 * L)
        ef = _edge_call(g, ef, edge_upd, 2048)
        agg2 = _sc_scatter(ef, dst)
        h0 = _node0_call(h0, agg2, node_upd, 1712)

    hrest = _hrest_call(hrest, node_upd, 2048)

    pooled0 = _pool_call(h0)
    pooledr = _pool_call(hrest)
    pooled = jnp.concatenate([pooled0, pooledr], axis=0)
    return _dec_call(pooled, dec)


# speedup vs baseline: 3.8502x; 1.2235x over previous
import functools

import jax
import jax.numpy as jnp
from jax import lax
from jax.experimental import pallas as pl
from jax.experimental.pallas import tpu as pltpu
from jax.experimental.pallas import tpu_sc as plsc

B = 8
V = 10242
E = 61440
L = 64
R = 16

NC = 2
NS = 16
NW = NC * NS
CH = 128
ZR = 641
VP = NS * ZR

F32 = jnp.float32


def _silu(x):
    return x * (1.0 / (1.0 + jnp.exp(-x)))


def _dot(a, b):
    return jnp.dot(a, b, preferred_element_type=F32)


def _full_spec(shape):
    return pl.BlockSpec(shape, lambda *_: tuple(0 for _ in shape))


def _mlp3_rows(x, w1, b1, w2, b2, w3, b3):
    t = _silu(_dot(x, w1) + b1)
    t = _silu(_dot(t, w2) + b2)
    return _dot(t, w3) + b3


def _enc_body(x_ref, w1, b1, w2, b2, w3, b3, out_ref):
    out_ref[...] = _mlp3_rows(x_ref[...], w1[...], b1[...], w2[...], b2[...],
                              w3[...], b3[...])


def _enc_call(x, ws):
    (w1, b1), (w2, b2), (w3, b3) = ws
    n, _ = x.shape
    return pl.pallas_call(
        _enc_body,
        out_shape=jax.ShapeDtypeStruct((n, w3.shape[1]), F32),
    )(x, w1, b1.reshape(1, -1), w2, b2.reshape(1, -1), w3, b3.reshape(1, -1))


def _rows_mlp_body(x_ref, w1, b1, w2, b2, w3, b3, out_ref):
    out_ref[...] = _mlp3_rows(x_ref[...], w1[...], b1[...], w2[...], b2[...],
                              w3[...], b3[...])


def _rows_mlp_call(x, ws, blk):
    (w1, b1), (w2, b2), (w3, b3) = ws
    n, k = x.shape
    grid = (pl.cdiv(n, blk),)
    return pl.pallas_call(
        _rows_mlp_body,
        grid=grid,
        in_specs=[
            pl.BlockSpec((blk, k), lambda i: (i, 0)),
            _full_spec(w1.shape), _full_spec((1, w1.shape[1])),
            _full_spec(w2.shape), _full_spec((1, w2.shape[1])),
            _full_spec(w3.shape), _full_spec((1, w3.shape[1])),
        ],
        out_specs=pl.BlockSpec((blk, w3.shape[1]), lambda i: (i, 0)),
        out_shape=jax.ShapeDtypeStruct((n, w3.shape[1]), F32),
    )(x, w1, b1.reshape(1, -1), w2, b2.reshape(1, -1), w3, b3.reshape(1, -1))


def _hinit_body(lat_ref, pos_ref, w1x, w1p, b1, w2, b2, w3, b3, out_ref):
    b = pl.program_id(0)
    sel = lax.broadcasted_iota(jnp.int32, (B, L), 0) == b
    xe = jnp.sum(jnp.where(sel, lat_ref[...], 0.0), axis=0, keepdims=True)
    t = _silu(_dot(xe, w1x[...]) + _dot(pos_ref[...], w1p[...]) + b1[...])
    t = _silu(_dot(t, w2[...]) + b2[...])
    out_ref[0] = _dot(t, w3[...]) + b3[...] + xe


def _hinit_call(lat, pos_emb, ws, blk):
    (w1, b1), (w2, b2), (w3, b3) = ws
    w1x, w1p = w1[:L], w1[L:]
    grid = (B, pl.cdiv(V, blk))
    return pl.pallas_call(
        _hinit_body,
        grid=grid,
        in_specs=[
            pl.BlockSpec((B, L), lambda b, v: (0, 0)),
            pl.BlockSpec((blk, L), lambda b, v: (v, 0)),
            _full_spec((L, L)), _full_spec((L, L)), _full_spec((1, L)),
            _full_spec((L, L)), _full_spec((1, L)),
            _full_spec((L, L)), _full_spec((1, L)),
        ],
        out_specs=pl.BlockSpec((1, blk, L), lambda b, v: (b, v, 0)),
        out_shape=jax.ShapeDtypeStruct((B, V, L), F32),
    )(lat, pos_emb, w1x, w1p, b1.reshape(1, -1), w2, b2.reshape(1, -1),
      w3, b3.reshape(1, -1))


def _edge_body(g_ref, ef_ref, w1g, w1e, b1, w2, b2, w3, b3, out_ref):
    ef = ef_ref[...]
    t = _silu(_dot(g_ref[...], w1g[...]) + _dot(ef, w1e[...]) + b1[...])
    t = _silu(_dot(t, w2[...]) + b2[...])
    out_ref[...] = _dot(t, w3[...]) + b3[...] + ef


def _edge_call(g, ef, ws, blk):
    (w1, b1), (w2, b2), (w3, b3) = ws
    w1g, w1e = w1[:2 * L], w1[2 * L:]
    grid = (E // blk,)
    return pl.pallas_call(
        _edge_body,
        grid=grid,
        in_specs=[
            pl.BlockSpec((blk, 2 * L), lambda i: (i, 0)),
            pl.BlockSpec((blk, L), lambda i: (i, 0)),
            _full_spec((2 * L, L)), _full_spec((L, L)), _full_spec((1, L)),
            _full_spec((L, L)), _full_spec((1, L)),
            _full_spec((L, L)), _full_spec((1, L)),
        ],
        out_specs=pl.BlockSpec((blk, L), lambda i: (i, 0)),
        out_shape=jax.ShapeDtypeStruct((E, L), F32),
    )(g, ef, w1g, w1e, b1.reshape(1, -1), w2, b2.reshape(1, -1),
      w3, b3.reshape(1, -1))


def _node0_body(h_ref, agg_ref, w1h, w1a, b1, w2, b2, w3, b3, out_ref):
    h = h_ref[...]
    agg = agg_ref[0] + agg_ref[1]
    t = _silu(_dot(h, w1h[...]) + _dot(agg, w1a[...]) + b1[...])
    t = _silu(_dot(t, w2[...]) + b2[...])
    out_ref[...] = _dot(t, w3[...]) + b3[...] + h


def _node0_call(h0, agg2, ws, blk):
    (w1, b1), (w2, b2), (w3, b3) = ws
    w1h, w1a = w1[:L], w1[L:]
    grid = (pl.cdiv(V, blk),)
    return pl.pallas_call(
        _node0_body,
        grid=grid,
        in_specs=[
            pl.BlockSpec((blk, L), lambda i: (i, 0)),
            pl.BlockSpec((NC, blk, L), lambda i: (0, i, 0)),
            _full_spec((L, L)), _full_spec((L, L)), _full_spec((1, L)),
            _full_spec((L, L)), _full_spec((1, L)),
            _full_spec((L, L)), _full_spec((1, L)),
        ],
        out_specs=pl.BlockSpec((blk, L), lambda i: (i, 0)),
        out_shape=jax.ShapeDtypeStruct((V, L), F32),
    )(h0, agg2, w1h, w1a, b1.reshape(1, -1), w2, b2.reshape(1, -1),
      w3, b3.reshape(1, -1))


def _hrest_body(h_ref, w1h, b1, w2, b2, w3, b3, out_ref):
    def rnd(_, h):
        t = _silu(_dot(h, w1h[...]) + b1[...])
        t = _silu(_dot(t, w2[...]) + b2[...])
        return _dot(t, w3[...]) + b3[...] + h

    out_ref[...] = lax.fori_loop(0, R, rnd, h_ref[...])


def _hrest_call(h, ws, blk):
    (w1, b1), (w2, b2), (w3, b3) = ws
    w1h = w1[:L]
    n = h.shape[0]
    grid = (pl.cdiv(n, blk),)
    return pl.pallas_call(
        _hrest_body,
        grid=grid,
        in_specs=[
            pl.BlockSpec((blk, L), lambda i: (i, 0)),
            _full_spec((L, L)), _full_spec((1, L)),
            _full_spec((L, L)), _full_spec((1, L)),
            _full_spec((L, L)), _full_spec((1, L)),
        ],
        out_specs=pl.BlockSpec((blk, L), lambda i: (i, 0)),
        out_shape=jax.ShapeDtypeStruct((n, L), F32),
    )(h, w1h, b1.reshape(1, -1), w2, b2.reshape(1, -1), w3, b3.reshape(1, -1))


def _pool_body(h_ref, out_ref):
    m = jnp.sum(h_ref[0], axis=0, keepdims=True) * (1.0 / V)
    out_ref[0] = jnp.broadcast_to(m, (8, L))


def _pool_call(h):
    k = h.shape[0] // V
    out = pl.pallas_call(
        _pool_body,
        grid=(k,),
        in_specs=[pl.BlockSpec((1, V, L), lambda i: (i, 0, 0))],
        out_specs=pl.BlockSpec((1, 8, L), lambda i: (i, 0, 0)),
        out_shape=jax.ShapeDtypeStruct((k, 8, L), F32),
    )(h.reshape(k, V, L))
    return out[:, 0, :]


def _dec_body(x_ref, w1, b1, w2, b2, w3, b3, out_ref):
    t = _silu(_dot(x_ref[...], w1[...]) + b1[...])
    t = _silu(_dot(t, w2[...]) + b2[...])
    out_ref[...] = _dot(t, w3[...]) + b3[...]


def _dec_call(x, ws):
    (w1, b1), (w2, b2), (w3, b3) = ws
    return pl.pallas_call(
        _dec_body,
        out_shape=jax.ShapeDtypeStruct((B, w3.shape[1]), F32),
    )(x, w1, b1.reshape(1, -1), w2, b2.reshape(1, -1), w3, b3.reshape(1, -1))


def _sc_mesh():
    return plsc.VectorSubcoreMesh(core_axis_name="c", subcore_axis_name="s",
                                  num_cores=NC, num_subcores=NS)


def _sc_gather(h0, idx2):
    n_chunks = (2 * E) // (NW * CH)
    per_w = (2 * E) // NW
    kb = 12
    glag = 6

    @functools.partial(
        pl.kernel,
        out_type=jax.ShapeDtypeStruct((2 * E, L), F32),
        mesh=_sc_mesh(),
        compiler_params=pltpu.CompilerParams(use_tc_tiling_on_sc=False),
        scratch_types=[
            pltpu.VMEM((per_w,), jnp.int32),
            pltpu.VMEM((kb, CH, L), F32),
            pltpu.SemaphoreType.DMA,
            pltpu.SemaphoreType.DMA,
        ],
    )
    def k(h0_hbm, idx_hbm, out_hbm, idx_v, bufs_v, gsem, wsem):
        wid = lax.axis_index("s") * NC + lax.axis_index("c")
        base = wid * per_w
        pltpu.sync_copy(idx_hbm.at[pl.ds(base, per_w)], idx_v)

        gd = [None] * n_chunks
        wd = [None] * n_chunks

        def issue_gather(j):
            gd[j] = pltpu.async_copy(
                h0_hbm.at[idx_v.at[pl.ds(j * CH, CH)]], bufs_v.at[j % kb], gsem)

        def issue_wb(j):
            wd[j] = pltpu.async_copy(
                bufs_v.at[j % kb], out_hbm.at[pl.ds(base + j * CH, CH)], wsem)

        for j in range(n_chunks):
            if j >= kb:
                wd[j - kb].wait()
            issue_gather(j)
            if j >= glag:
                gd[j - glag].wait()
                issue_wb(j - glag)
        for j in range(n_chunks - glag, n_chunks):
            gd[j].wait()
            issue_wb(j)
        for j in range(max(0, n_chunks - kb), n_chunks):
            if wd[j] is not None:
                wd[j].wait()

    return k(h0, idx2)


def _sc_scatter(ef, dst3):
    n_chunks = E // (NW * CH)
    per_w = E // NW
    kb = 4

    @functools.partial(
        pl.kernel,
        out_type=jax.ShapeDtypeStruct((NC, VP, L), F32),
        mesh=_sc_mesh(),
        compiler_params=pltpu.CompilerParams(use_tc_tiling_on_sc=False),
        scratch_types=[
            pltpu.VMEM((n_chunks, CH), jnp.int32),
            pltpu.VMEM((kb, CH, L), F32),
            pltpu.VMEM((ZR, L), F32),
            pltpu.VMEM_SHARED((VP, L), F32),
            pltpu.SemaphoreType.DMA,
            pltpu.SemaphoreType.DMA,
        ],
    )
    def k(ef_hbm, dst_hbm, agg_hbm, idx_v, bufs_v, zbuf_v, agg_sh, lsem, ssem):
        c = lax.axis_index("c")
        s = lax.axis_index("s")
        wid = s * NC + c
        base = wid * per_w
        rows0 = s * ZR

        idxd = pltpu.async_copy(dst_hbm.at[wid], idx_v, lsem)

        z16 = jnp.zeros((16,), F32)

        def zrow(i, carry):
            for j in range(L // 16):
                zbuf_v[i, pl.ds(j * 16, 16)] = z16
            return carry

        lax.fori_loop(0, ZR, zrow, 0)
        pltpu.sync_copy(zbuf_v, agg_sh.at[pl.ds(rows0, ZR)])
        idxd.wait()
        plsc.subcore_barrier()

        ld = [None] * n_chunks
        sd = [None] * n_chunks

        def issue_load(j):
            ld[j] = pltpu.async_copy(
                ef_hbm.at[pl.ds(base + j * CH, CH)], bufs_v.at[j % kb], lsem)

        def issue_add(j):
            sd[j] = pltpu.async_copy(
                bufs_v.at[j % kb], agg_sh.at[idx_v.at[j]], ssem, add=True)

        for j in range(n_chunks):
            if j >= kb:
                sd[j - kb].wait()
            issue_load(j)
            if j >= 2:
                ld[j - 2].wait()
                issue_add(j - 2)
        for j in range(n_chunks - 2, n_chunks):
            ld[j].wait()
            issue_add(j)
        for j in range(max(0, n_chunks - kb), n_chunks):
            sd[j].wait()
        plsc.subcore_barrier()

        pltpu.sync_copy(agg_sh.at[pl.ds(rows0, ZR)], zbuf_v)
        pltpu.sync_copy(zbuf_v, agg_hbm.at[c, pl.ds(rows0, ZR)])

    return k(ef, dst3)


def kernel(x, node_pos, edge_feat4, enc, pos_mlp, ns_mlp, edge_map,
           node_upd, edge_upd, dec, edge_index):
    lat = _enc_call(x, enc)

    npad = jnp.pad(node_pos, ((0, 0), (0, 5)))
    pw1, pb1 = pos_mlp[0]
    pos_ws = [(jnp.pad(pw1, ((0, 5), (0, 0))), pb1)] + list(pos_mlp[1:])
    pos_emb = _rows_mlp_call(npad, pos_ws, 2048)

    h = _hinit_call(lat, pos_emb, ns_mlp, 2048)
    h0 = h[0]
    hrest = h[1:].reshape((B - 1) * V, L)

    epad = jnp.pad(edge_feat4, ((0, 0), (0, 4)))
    ew1, eb1 = edge_map[0]
    edge_ws = [(jnp.pad(ew1, ((0, 4), (0, 0))), eb1)] + list(edge_map[1:])
    ef = _rows_mlp_call(epad, edge_ws, 2048)

    src = edge_index[0]
    dst = edge_index[1]
    idx2 = jnp.stack([src, dst], axis=1).reshape(2 * E)
    dst3 = dst.reshape(NW, E // (NW * CH), CH)

    for _ in range(R):
        g = _sc_gather(h0, idx2).reshape(E, 2 * L)
        ef = _edge_call(g, ef, edge_upd, 2048)
        agg2 = _sc_scatter(ef, dst3)
        h0 = _node0_call(h0, agg2, node_upd, 1712)

    hrest = _hrest_call(hrest, node_upd, 2048)

    pooled0 = _pool_call(h0)
    pooledr = _pool_call(hrest)
    pooled = jnp.concatenate([pooled0, pooledr], axis=0)
    return _dec_call(pooled, dec)
